# baseline (speedup 1.0000x reference)
import functools

import jax
import jax.numpy as jnp
from jax import lax
from jax.experimental import pallas as pl
from jax.experimental.pallas import tpu as pltpu
from jax.experimental.pallas import tpu_sc as plsc

D = 64
DH = 32
CHUNK = 25000
CP = 25088
TS = CP // 16
ZB = 224
G = 512
NCHUNK = 4


def _prep_edges(lap_rows, lap_cols, lap_vals, half, ep2):
    parts = []
    for a in (lap_rows, lap_cols,
              jax.lax.bitcast_convert_type(lap_vals, jnp.int32)):
        a2 = a.reshape(2, half)
        a2 = jnp.pad(a2, ((0, 0), (0, ep2 - half)))
        parts.append(a2.reshape(2, ep2 // 128, 128))
    return jnp.stack(parts, axis=2)


def _spmm(edata, ego_lo, ego_hi, jblk):
    mesh = plsc.VectorSubcoreMesh(core_axis_name="c", subcore_axis_name="s")

    @functools.partial(
        pl.kernel,
        out_type=jax.ShapeDtypeStruct((NCHUNK, 2, CP, DH), jnp.float32),
        mesh=mesh,
        scratch_types=[
            pltpu.VMEM((2, 4, 3, 128), jnp.int32),
            pltpu.VMEM((4, 128), jnp.float32),
            pltpu.VMEM((4, 128), jnp.int32),
            pltpu.VMEM((2, G, DH), jnp.float32),
            pltpu.VMEM((ZB, DH), jnp.float32),
            pltpu.VMEM_SHARED((CP, DH), jnp.float32),
            pltpu.SemaphoreType.DMA,
            pltpu.SemaphoreType.DMA,
        ],
        compiler_params=pltpu.CompilerParams(use_tc_tiling_on_sc=False,
                                             needs_layout_passes=False),
    )
    def k(edata_hbm, elo_hbm, ehi_hbm, out_hbm,
          ebuf, vbuf, ibuf, gbuf, zbuf, acc, sem_e, sem_g):
        c = lax.axis_index("c")
        s = lax.axis_index("s")

        def zz(i, carry):
            for jj in range(DH // 16):
                zbuf[i, pl.ds(jj * 16, 16)] = jnp.zeros((16,), jnp.float32)
            return carry
        lax.fori_loop(0, ZB, zz, 0)

        for p in range(2):
            for f in range(2):
                ch = 2 * p + c
                lo = ch * CHUNK
                ego_hbm = elo_hbm if f == 0 else ehi_hbm

                def stage_src(j):
                    return edata_hbm.at[p, pl.ds(j * 64 + s * 4, 4)]

                def fire_gathers(b):
                    for q in range(4):
                        pltpu.async_copy(ego_hbm.at[ebuf.at[b, q, 1]],
                                         gbuf.at[b, pl.ds(q * 128, 128)],
                                         sem_g)

                def wait_gathers(b):
                    for q in range(4):
                        pltpu.make_async_copy(
                            ego_hbm.at[ebuf.at[b, q, 1]],
                            gbuf.at[b, pl.ds(q * 128, 128)],
                            sem_g).wait()

                for kq in range(TS // ZB):
                    pltpu.sync_copy(zbuf, acc.at[pl.ds(s * TS + kq * ZB, ZB)])
                plsc.subcore_barrier()

                pltpu.sync_copy(stage_src(0), ebuf.at[0])
                pltpu.async_copy(stage_src(1), ebuf.at[1], sem_e)
                fire_gathers(0)

                def blk2(j2, carry):
                    for bcur in range(2):
                        jcur = j2 * 2 + bcur
                        bnext = 1 - bcur
                        for q in range(4):
                            def msk(i, carry2):
                                sl = pl.ds(i * 16, 16)
                                rv = ebuf[bcur, q, 0, sl]
                                vv = plsc.bitcast(ebuf[bcur, q, 2, sl],
                                                  jnp.float32)
                                m = (rv >= lo) & (rv < lo + CHUNK)
                                vbuf[q, sl] = jnp.where(m, vv, 0.0)
                                ibuf[q, sl] = jnp.where(m, rv - lo, 0)
                                return carry2
                            lax.fori_loop(0, 8, msk, 0)
                        wait_gathers(bcur)
                        pltpu.async_copy(stage_src(jcur + 2), ebuf.at[bcur],
                                         sem_e)
                        for q in range(4):
                            def erow16(i, carry2):
                                sv = vbuf[q, pl.ds(i * 16, 16)]
                                for k16 in range(16):
                                    sc = sv[k16]
                                    r = q * 128 + i * 16 + k16
                                    for jj in range(DH // 16):
                                        sl = pl.ds(jj * 16, 16)
                                        gbuf[bcur, r, sl] = \
                                            gbuf[bcur, r, sl] * sc
                                return carry2
                            lax.fori_loop(0, 8, erow16, 0)
                        for q in range(4):
                            pltpu.sync_copy(gbuf.at[bcur, pl.ds(q * 128, 128)],
                                            acc.at[ibuf.at[q]], add=True)
                        pltpu.make_async_copy(stage_src(jcur + 1),
                                              ebuf.at[bnext], sem_e).wait()
                        fire_gathers(bnext)
                    return carry
                lax.fori_loop(0, jblk // 2, blk2, 0)

                wait_gathers(0)
                pltpu.make_async_copy(stage_src(jblk + 1), ebuf.at[1],
                                      sem_e).wait()

                plsc.subcore_barrier()
                pltpu.sync_copy(acc.at[pl.ds(s * TS, TS)],
                                out_hbm.at[ch, f, pl.ds(s * TS, TS)])

    return k(edata, ego_lo, ego_hi)


def _dense(msg_p, ego_s, w1, b1, w2, b2, n_nodes):
    br = 1000
    jgrid = CHUNK // br

    def body(mlo_ref, mhi_ref, elo_ref, ehi_ref,
             w1_ref, b1_ref, w2_ref, b2_ref, eo_ref, no_ref):
        egos = jnp.concatenate([elo_ref[0], ehi_ref[0]], axis=1)
        msg = jnp.concatenate([mlo_ref[0, 0], mhi_ref[0, 0]], axis=1)
        side = msg + egos
        dn = (((1,), (0,)), ((), ()))
        a = lax.dot_general(side, w1_ref[...], dn,
                            preferred_element_type=jnp.float32) + b1_ref[...]
        se = jnp.maximum(a, 0.2 * a)
        b = lax.dot_general(egos * side, w2_ref[...], dn,
                            preferred_element_type=jnp.float32) + b2_ref[...]
        be = jnp.maximum(b, 0.2 * b)
        e = se + be
        eo_ref[0] = e[:, :DH]
        eo_ref[1] = e[:, DH:]
        nr = jnp.sqrt(jnp.sum(e * e, axis=1, keepdims=True)) + 1e-12
        no_ref[...] = e / nr

    return pl.pallas_call(
        body,
        grid=(NCHUNK, jgrid),
        in_specs=[
            pl.BlockSpec((1, 1, br, DH), lambda p, j: (p, 0, j, 0)),
            pl.BlockSpec((1, 1, br, DH), lambda p, j: (p, 1, j, 0)),
            pl.BlockSpec((1, br, DH), lambda p, j: (0, p * jgrid + j, 0)),
            pl.BlockSpec((1, br, DH), lambda p, j: (1, p * jgrid + j, 0)),
            pl.BlockSpec((D, D), lambda p, j: (0, 0)),
            pl.BlockSpec((1, D), lambda p, j: (0, 0)),
            pl.BlockSpec((D, D), lambda p, j: (0, 0)),
            pl.BlockSpec((1, D), lambda p, j: (0, 0)),
        ],
        out_specs=[
            pl.BlockSpec((2, br, DH), lambda p, j: (0, p * jgrid + j, 0)),
            pl.BlockSpec((br, D), lambda p, j: (p * jgrid + j, 0)),
        ],
        out_shape=(jax.ShapeDtypeStruct((2, n_nodes, DH), jnp.float32),
                   jax.ShapeDtypeStruct((n_nodes, D), jnp.float32)),
    )(msg_p, msg_p, ego_s, ego_s, w1, b1.reshape(1, D), w2, b2.reshape(1, D))


def _final_gather(users, pos_items, neg_items, tables, n_user, batch):
    mesh = plsc.VectorSubcoreMesh(core_axis_name="c", subcore_axis_name="s")
    per_w = batch // 32
    width = D * len(tables)

    @functools.partial(
        pl.kernel,
        out_type=tuple(jax.ShapeDtypeStruct((batch, width), jnp.float32)
                       for _ in range(3)),
        mesh=mesh,
        scratch_types=[
            pltpu.VMEM((per_w,), jnp.int32),
            pltpu.VMEM((per_w, D), jnp.float32),
            pltpu.SemaphoreType.DMA,
        ],
        compiler_params=pltpu.CompilerParams(use_tc_tiling_on_sc=False),
    )
    def k(users_h, pos_h, neg_h, t0, t1, t2, t3, o0, o1, o2,
          ibuf, gbuf, sem):
        c = lax.axis_index("c")
        s = lax.axis_index("s")
        wid = s * 2 + c
        base = wid * per_w
        for src, off, out in ((users_h, 0, o0), (pos_h, n_user, o1),
                              (neg_h, n_user, o2)):
            pltpu.sync_copy(src.at[pl.ds(base, per_w)], ibuf)
            if off:
                def addoff(i, carry):
                    sl = pl.ds(i * 16, 16)
                    ibuf[sl] = ibuf[sl] + off
                    return carry
                lax.fori_loop(0, per_w // 16, addoff, 0)
            for t, tab in enumerate((t0, t1, t2, t3)):
                pltpu.async_copy(tab.at[ibuf], gbuf, sem).wait()
                pltpu.sync_copy(gbuf, out.at[pl.ds(base, per_w),
                                             pl.ds(t * D, D)])

    return k(users, pos_items, neg_items, *tables)


def kernel(users, pos_items, neg_items, lap_rows, lap_cols, lap_vals,
           user_emb, item_emb, W1, b1, W2, b2):
    n_user = user_emb.shape[0]
    n_nodes = n_user + item_emb.shape[0]
    nnz = lap_rows.shape[0]
    half = nnz // 2
    batch = users.shape[0]
    jblk = -(-half // (16 * 4 * 128))
    jblk += jblk % 2
    ep2 = (jblk + 2) * 16 * 4 * 128

    edata = _prep_edges(lap_rows, lap_cols, lap_vals, half, ep2)

    ego0 = jnp.concatenate([user_emb, item_emb], axis=0)
    ego_s = jnp.stack([ego0[:, :DH], ego0[:, DH:]], axis=0)
    tables = [ego0]
    for l in range(len(W1)):
        msg_p = _spmm(edata, ego_s[0], ego_s[1], jblk)
        ego_s, nrm = _dense(msg_p, ego_s, W1[l], b1[l], W2[l], b2[l], n_nodes)
        tables.append(nrm)
    return _final_gather(users, pos_items, neg_items, tables, n_user, batch)


# speedup vs baseline: 1.4631x; 1.4631x over previous
import functools

import jax
import jax.numpy as jnp
from jax import lax
from jax.experimental import pallas as pl
from jax.experimental.pallas import tpu as pltpu
from jax.experimental.pallas import tpu_sc as plsc

D = 64
NCH = 8
CH = 12500
CP = 12544
TS = CP // 16
TSL = CH - 15 * TS
ZB = 224
G = 512
BUFW = 1040


def _compact(edata_rc, edata_v, g2, nbw):
    mesh = plsc.VectorSubcoreMesh(core_axis_name="c", subcore_axis_name="s")
    gwg = 2 * g2 // 32
    wb = gwg // 4

    @functools.partial(
        pl.kernel,
        out_type=(jax.ShapeDtypeStruct((NCH * 32 * nbw, G), jnp.int32),
                  jax.ShapeDtypeStruct((NCH * 32 * nbw, G), jnp.int32),
                  jax.ShapeDtypeStruct((NCH * 32 * nbw, G), jnp.float32),
                  jax.ShapeDtypeStruct((32, 128), jnp.int32)),
        mesh=mesh,
        scratch_types=[
            pltpu.VMEM((4, 2, 128), jnp.int32),
            pltpu.VMEM((4, 128), jnp.float32),
        ] + [pltpu.VMEM((BUFW,), jnp.int32) for _ in range(NCH)]
          + [pltpu.VMEM((BUFW,), jnp.int32) for _ in range(NCH)]
          + [pltpu.VMEM((BUFW,), jnp.float32) for _ in range(NCH)]
          + [pltpu.VMEM((128,), jnp.int32)],
        compiler_params=pltpu.CompilerParams(use_tc_tiling_on_sc=False,
                                             needs_layout_passes=False),
    )
    def k(rc_hbm, v_hbm, col_out, idx_out, v_out, cnt_out, rcbuf, vstg,
          *bufs):
        bcol = bufs[0:NCH]
        bidx = bufs[NCH:2 * NCH]
        bval = bufs[2 * NCH:3 * NCH]
        cbuf = bufs[3 * NCH]
        iota16 = lax.iota(jnp.int32, 16)
        c = lax.axis_index("c")
        s = lax.axis_index("s")
        wid = s * 2 + c
        p = wid // 16
        gbase = (wid % 16) * gwg

        def blk(j, carry):
            pltpu.sync_copy(rc_hbm.at[p, pl.ds(gbase + j * 4, 4)], rcbuf)
            pltpu.sync_copy(v_hbm.at[p, pl.ds(gbase + j * 4, 4)], vstg)

            def grp(i, carry2):
                q = i // 8
                o = (i % 8) * 16
                rv = rcbuf[q, 0, pl.ds(o, 16)]
                cv = rcbuf[q, 1, pl.ds(o, 16)]
                vv = vstg[q, pl.ds(o, 16)]
                new = []
                for b in range(NCH):
                    m = (rv >= b * CH) & (rv < (b + 1) * CH)
                    il = rv - b * CH
                    w = carry2[b]
                    plsc.store_compressed(bcol[b].at[pl.ds(w, 16)], cv,
                                          mask=m)
                    plsc.store_compressed(bidx[b].at[pl.ds(w, 16)], il,
                                          mask=m)
                    plsc.store_compressed(bval[b].at[pl.ds(w, 16)], vv,
                                          mask=m)
                    pc = plsc.all_reduce_population_count(m)
                    if pc.shape:
                        pc = pc[0]
                    new.append((w + pc).astype(jnp.int32))
                return tuple(new) + carry2[NCH:]
            carry = lax.fori_loop(0, 32, grp, carry)

            new_w, new_h = [], []
            for b in range(NCH):
                w, h = carry[b], carry[NCH + b]
                fl = w >= G

                @pl.when(fl)
                def _():
                    row = (b * 32 + wid) * nbw + h
                    pltpu.sync_copy(bcol[b].at[pl.ds(0, G)], col_out.at[row])
                    pltpu.sync_copy(bidx[b].at[pl.ds(0, G)], idx_out.at[row])
                    pltpu.sync_copy(bval[b].at[pl.ds(0, G)], v_out.at[row])
                    def mv(i, carry3):
                        sl_src = pl.ds(G + i * 16, 16)
                        sl_dst = pl.ds(i * 16, 16)
                        bcol[b][sl_dst] = bcol[b][sl_src]
                        bidx[b][sl_dst] = bidx[b][sl_src]
                        bval[b][sl_dst] = bval[b][sl_src]
                        return carry3
                    lax.fori_loop(0, G // 16, mv, 0)
                new_w.append(jnp.where(fl, w - G, w))
                new_h.append(h + fl.astype(jnp.int32))
            return tuple(new_w) + tuple(new_h)

        carry = lax.fori_loop(0, wb, blk, (jnp.int32(0),) * (2 * NCH))

        cvec = jnp.zeros((16,), jnp.int32)
        for b in range(NCH):
            w, h = carry[b], carry[NCH + b]
            row = (b * 32 + wid) * nbw + h
            pltpu.sync_copy(bcol[b].at[pl.ds(0, G)], col_out.at[row])
            pltpu.sync_copy(bidx[b].at[pl.ds(0, G)], idx_out.at[row])
            pltpu.sync_copy(bval[b].at[pl.ds(0, G)], v_out.at[row])
            cvec = jnp.where(iota16 == b, h * G + w, cvec)
        for i8 in range(8):
            cbuf[pl.ds(i8 * 16, 16)] = cvec if i8 == 0 else \
                jnp.zeros((16,), jnp.int32)
        pltpu.sync_copy(cbuf, cnt_out.at[wid])

    return k(edata_rc, edata_v)


def _spmm(comp_col, comp_idx, comp_v, counts, ego, n_nodes, nbw):
    mesh = plsc.VectorSubcoreMesh(core_axis_name="c", subcore_axis_name="s")

    @functools.partial(
        pl.kernel,
        out_type=jax.ShapeDtypeStruct((n_nodes, D), jnp.float32),
        mesh=mesh,
        scratch_types=[
            pltpu.VMEM((G,), jnp.int32),
            pltpu.VMEM((G,), jnp.int32),
            pltpu.VMEM((G,), jnp.float32),
            pltpu.VMEM((4, 128), jnp.int32),
            pltpu.VMEM((4, 128), jnp.float32),
            pltpu.VMEM((4, 128), jnp.int32),
            pltpu.VMEM((G, D), jnp.float32),
            pltpu.VMEM((ZB, D), jnp.float32),
            pltpu.VMEM((128,), jnp.int32),
            pltpu.VMEM_SHARED((CP, D), jnp.float32),
            pltpu.SemaphoreType.DMA,
        ],
        compiler_params=pltpu.CompilerParams(use_tc_tiling_on_sc=False,
                                             needs_layout_passes=False),
    )
    def k(col_hbm, idx_hbm, v_hbm, cnt_hbm, ego_hbm, out_hbm,
          ccol, cidx, vstg, colb, vbuf, ibuf, gbuf, zbuf, cbuf, acc, sem):
        iota16 = lax.iota(jnp.int32, 16)
        c = lax.axis_index("c")
        s = lax.axis_index("s")

        def zz(i, carry):
            for jj in range(D // 16):
                zbuf[i, pl.ds(jj * 16, 16)] = jnp.zeros((16,), jnp.float32)
            return carry
        lax.fori_loop(0, ZB, zz, 0)

        for pp in range(4):
            ch = 2 * pp + c
            for kq in range(3):
                pltpu.sync_copy(zbuf, acc.at[pl.ds(s * TS + kq * ZB, ZB)])
            pltpu.sync_copy(zbuf.at[pl.ds(0, TS - 3 * ZB)],
                            acc.at[pl.ds(s * TS + 3 * ZB, TS - 3 * ZB)])
            plsc.subcore_barrier()

            for seg in range(1):
                w = jnp.where(ch >= NCH // 2, 16, 0) + s
                pltpu.sync_copy(cnt_hbm.at[w], cbuf)
                cvec = cbuf[pl.ds(0, 16)]
                cnt = jnp.int32(0)
                for kk in range(NCH):
                    cnt = jnp.where(ch == kk, cvec[kk], cnt)
                nb = jnp.maximum((cnt + G - 1) // G, 1)

                def blk(j, carry):
                    row = (ch * 32 + w) * nbw + j
                    pltpu.sync_copy(col_hbm.at[row], ccol)
                    pltpu.sync_copy(idx_hbm.at[row], cidx)
                    pltpu.sync_copy(v_hbm.at[row], vstg)
                    base = j * G
                    for q in range(4):
                        def msk(i, carry2):
                            sl = pl.ds(q * 128 + i * 16, 16)
                            lane = base + q * 128 + i * 16 + iota16
                            valid = lane < cnt
                            so = pl.ds(i * 16, 16)
                            colb[q, so] = jnp.where(valid, ccol[sl], 0)
                            vbuf[q, so] = jnp.where(valid, vstg[sl], 0.0)
                            ibuf[q, so] = jnp.where(valid, cidx[sl], 0)
                            return carry2
                        lax.fori_loop(0, 8, msk, 0)
                    gds = [
                        pltpu.async_copy(ego_hbm.at[colb.at[q]],
                                         gbuf.at[pl.ds(q * 128, 128)], sem)
                        for q in range(4)
                    ]
                    for gd in gds:
                        gd.wait()
                    for q in range(4):
                        def erow16(i, carry2):
                            sv = vbuf[q, pl.ds(i * 16, 16)]
                            for k16 in range(16):
                                sc = sv[k16]
                                r = q * 128 + i * 16 + k16
                                for jj in range(D // 16):
                                    sl = pl.ds(jj * 16, 16)
                                    gbuf[r, sl] = gbuf[r, sl] * sc
                            return carry2
                        lax.fori_loop(0, 8, erow16, 0)
                    for q in range(4):
                        pltpu.sync_copy(gbuf.at[pl.ds(q * 128, 128)],
                                        acc.at[ibuf.at[q]], add=True)
                    return carry
                lax.fori_loop(0, nb, blk, 0)

            plsc.subcore_barrier()
            base = ch * CH + s * TS

            @pl.when(s < 15)
            def _():
                pltpu.sync_copy(acc.at[pl.ds(s * TS, TS)],
                                out_hbm.at[pl.ds(base, TS)])

            @pl.when(s == 15)
            def _():
                pltpu.sync_copy(acc.at[pl.ds(s * TS, TSL)],
                                out_hbm.at[pl.ds(base, TSL)])

    return k(comp_col, comp_idx, comp_v, counts, ego)


def _dense(msg, ego, w1, b1, w2, b2, n_nodes):
    br = 1000
    grid = n_nodes // br

    def body(msg_ref, ego_ref, w1_ref, b1_ref, w2_ref, b2_ref,
             eo_ref, no_ref):
        egos = ego_ref[...]
        side = msg_ref[...] + egos
        dn = (((1,), (0,)), ((), ()))
        a = lax.dot_general(side, w1_ref[...], dn,
                            preferred_element_type=jnp.float32) + b1_ref[...]
        se = jnp.maximum(a, 0.2 * a)
        b = lax.dot_general(egos * side, w2_ref[...], dn,
                            preferred_element_type=jnp.float32) + b2_ref[...]
        be = jnp.maximum(b, 0.2 * b)
        e = se + be
        eo_ref[...] = e
        nr = jnp.sqrt(jnp.sum(e * e, axis=1, keepdims=True)) + 1e-12
        no_ref[...] = e / nr

    return pl.pallas_call(
        body,
        grid=(grid,),
        in_specs=[
            pl.BlockSpec((br, D), lambda i: (i, 0)),
            pl.BlockSpec((br, D), lambda i: (i, 0)),
            pl.BlockSpec((D, D), lambda i: (0, 0)),
            pl.BlockSpec((1, D), lambda i: (0, 0)),
            pl.BlockSpec((D, D), lambda i: (0, 0)),
            pl.BlockSpec((1, D), lambda i: (0, 0)),
        ],
        out_specs=[pl.BlockSpec((br, D), lambda i: (i, 0))] * 2,
        out_shape=(jax.ShapeDtypeStruct((n_nodes, D), jnp.float32),
                   jax.ShapeDtypeStruct((n_nodes, D), jnp.float32)),
    )(msg, ego, w1, b1.reshape(1, D), w2, b2.reshape(1, D))


def _final_gather(users, pos_items, neg_items, tables, n_user, batch):
    mesh = plsc.VectorSubcoreMesh(core_axis_name="c", subcore_axis_name="s")
    per_w = batch // 32
    width = D * len(tables)

    @functools.partial(
        pl.kernel,
        out_type=tuple(jax.ShapeDtypeStruct((batch, width), jnp.float32)
                       for _ in range(3)),
        mesh=mesh,
        scratch_types=[
            pltpu.VMEM((per_w,), jnp.int32),
            pltpu.VMEM((per_w, D), jnp.float32),
            pltpu.SemaphoreType.DMA,
        ],
        compiler_params=pltpu.CompilerParams(use_tc_tiling_on_sc=False,
                                             needs_layout_passes=False),
    )
    def k(users_h, pos_h, neg_h, t0, t1, t2, t3, o0, o1, o2,
          ibuf, gbuf, sem):
        c = lax.axis_index("c")
        s = lax.axis_index("s")
        wid = s * 2 + c
        base = wid * per_w
        for src, off, out in ((users_h, 0, o0), (pos_h, n_user, o1),
                              (neg_h, n_user, o2)):
            pltpu.sync_copy(src.at[pl.ds(base, per_w)], ibuf)
            if off:
                def addoff(i, carry):
                    sl = pl.ds(i * 16, 16)
                    ibuf[sl] = ibuf[sl] + off
                    return carry
                lax.fori_loop(0, per_w // 16, addoff, 0)
            for t, tab in enumerate((t0, t1, t2, t3)):
                pltpu.async_copy(tab.at[ibuf], gbuf, sem).wait()
                pltpu.sync_copy(gbuf, out.at[pl.ds(base, per_w),
                                             pl.ds(t * D, D)])

    return k(users, pos_items, neg_items, *tables)


def kernel(users, pos_items, neg_items, lap_rows, lap_cols, lap_vals,
           user_emb, item_emb, W1, b1, W2, b2):
    n_user = user_emb.shape[0]
    n_nodes = n_user + item_emb.shape[0]
    nnz = lap_rows.shape[0]
    half = nnz // 2
    batch = users.shape[0]
    jblk = -(-half // (16 * 4 * 128))
    jblk += jblk % 2
    ep2 = (jblk + 2) * 16 * 4 * 128
    g2 = ep2 // 128
    ew = 2 * ep2 // 32
    nbw = ew // G + 2

    def prep(a):
        a2 = a.reshape(2, half)
        return jnp.pad(a2, ((0, 0), (0, ep2 - half))).reshape(2, g2, 128)

    edata_rc = jnp.stack([prep(lap_rows), prep(lap_cols)], axis=2)
    edata_v = prep(lap_vals)

    comp_col, comp_idx, comp_v, counts = _compact(edata_rc, edata_v, g2, nbw)

    ego = jnp.concatenate([user_emb, item_emb], axis=0)
    tables = [ego]
    for l in range(len(W1)):
        msg = _spmm(comp_col, comp_idx, comp_v, counts, ego, n_nodes, nbw)
        ego, nrm = _dense(msg, ego, W1[l], b1[l], W2[l], b2[l], n_nodes)
        tables.append(nrm)
    return _final_gather(users, pos_items, neg_items, tables, n_user, batch)


# speedup vs baseline: 1.5411x; 1.0533x over previous
import functools

import jax
import jax.numpy as jnp
from jax import lax
from jax.experimental import pallas as pl
from jax.experimental.pallas import tpu as pltpu
from jax.experimental.pallas import tpu_sc as plsc

D = 64
NCH = 8
CH = 12500
CP = 12544
TS = CP // 16
TSL = CH - 15 * TS
ZB = 224
G = 512
BUFW = 1040


def _compact(edata_rc, edata_v, g2, nbw):
    mesh = plsc.VectorSubcoreMesh(core_axis_name="c", subcore_axis_name="s")
    gwg = 2 * g2 // 32
    wb = gwg // 4

    @functools.partial(
        pl.kernel,
        out_type=(jax.ShapeDtypeStruct((NCH * 32 * nbw, G), jnp.int32),
                  jax.ShapeDtypeStruct((NCH * 32 * nbw, G), jnp.int32),
                  jax.ShapeDtypeStruct((NCH * 32 * nbw, G), jnp.float32),
                  jax.ShapeDtypeStruct((32, 128), jnp.int32)),
        mesh=mesh,
        scratch_types=[
            pltpu.VMEM((4, 2, 128), jnp.int32),
            pltpu.VMEM((4, 128), jnp.float32),
        ] + [pltpu.VMEM((BUFW,), jnp.int32) for _ in range(NCH)]
          + [pltpu.VMEM((BUFW,), jnp.int32) for _ in range(NCH)]
          + [pltpu.VMEM((BUFW,), jnp.float32) for _ in range(NCH)]
          + [pltpu.VMEM((128,), jnp.int32)],
        compiler_params=pltpu.CompilerParams(use_tc_tiling_on_sc=False,
                                             needs_layout_passes=False),
    )
    def k(rc_hbm, v_hbm, col_out, idx_out, v_out, cnt_out, rcbuf, vstg,
          *bufs):
        bcol = bufs[0:NCH]
        bidx = bufs[NCH:2 * NCH]
        bval = bufs[2 * NCH:3 * NCH]
        cbuf = bufs[3 * NCH]
        iota16 = lax.iota(jnp.int32, 16)
        c = lax.axis_index("c")
        s = lax.axis_index("s")
        wid = s * 2 + c
        p = wid // 16
        gbase = (wid % 16) * gwg

        def blk(j, carry):
            pltpu.sync_copy(rc_hbm.at[p, pl.ds(gbase + j * 4, 4)], rcbuf)
            pltpu.sync_copy(v_hbm.at[p, pl.ds(gbase + j * 4, 4)], vstg)

            def grp(i, carry2):
                q = i // 8
                o = (i % 8) * 16
                rv = rcbuf[q, 0, pl.ds(o, 16)]
                cv = rcbuf[q, 1, pl.ds(o, 16)]
                vv = vstg[q, pl.ds(o, 16)]
                new = []
                for b in range(NCH):
                    m = (rv >= b * CH) & (rv < (b + 1) * CH)
                    il = rv - b * CH
                    w = carry2[b]
                    plsc.store_compressed(bcol[b].at[pl.ds(w, 16)], cv,
                                          mask=m)
                    plsc.store_compressed(bidx[b].at[pl.ds(w, 16)], il,
                                          mask=m)
                    plsc.store_compressed(bval[b].at[pl.ds(w, 16)], vv,
                                          mask=m)
                    pc = plsc.all_reduce_population_count(m)
                    if pc.shape:
                        pc = pc[0]
                    new.append((w + pc).astype(jnp.int32))
                return tuple(new) + carry2[NCH:]
            carry = lax.fori_loop(0, 32, grp, carry)

            new_w, new_h = [], []
            for b in range(NCH):
                w, h = carry[b], carry[NCH + b]
                fl = w >= G

                @pl.when(fl)
                def _():
                    row = (b * 32 + wid) * nbw + h
                    pltpu.sync_copy(bcol[b].at[pl.ds(0, G)], col_out.at[row])
                    pltpu.sync_copy(bidx[b].at[pl.ds(0, G)], idx_out.at[row])
                    pltpu.sync_copy(bval[b].at[pl.ds(0, G)], v_out.at[row])
                    def mv(i, carry3):
                        sl_src = pl.ds(G + i * 16, 16)
                        sl_dst = pl.ds(i * 16, 16)
                        bcol[b][sl_dst] = bcol[b][sl_src]
                        bidx[b][sl_dst] = bidx[b][sl_src]
                        bval[b][sl_dst] = bval[b][sl_src]
                        return carry3
                    lax.fori_loop(0, G // 16, mv, 0)
                new_w.append(jnp.where(fl, w - G, w))
                new_h.append(h + fl.astype(jnp.int32))
            return tuple(new_w) + tuple(new_h)

        carry = lax.fori_loop(0, wb, blk, (jnp.int32(0),) * (2 * NCH))

        cvec = jnp.zeros((16,), jnp.int32)
        for b in range(NCH):
            w, h = carry[b], carry[NCH + b]
            row = (b * 32 + wid) * nbw + h
            pltpu.sync_copy(bcol[b].at[pl.ds(0, G)], col_out.at[row])
            pltpu.sync_copy(bidx[b].at[pl.ds(0, G)], idx_out.at[row])
            pltpu.sync_copy(bval[b].at[pl.ds(0, G)], v_out.at[row])
            cvec = jnp.where(iota16 == b, h * G + w, cvec)
        for i8 in range(8):
            cbuf[pl.ds(i8 * 16, 16)] = cvec if i8 == 0 else \
                jnp.zeros((16,), jnp.int32)
        pltpu.sync_copy(cbuf, cnt_out.at[wid])

    return k(edata_rc, edata_v)


def _spmm(comp_col, comp_idx, comp_v, counts, ego, n_nodes, nbw):
    mesh = plsc.VectorSubcoreMesh(core_axis_name="c", subcore_axis_name="s")

    @functools.partial(
        pl.kernel,
        out_type=jax.ShapeDtypeStruct((n_nodes, D), jnp.float32),
        mesh=mesh,
        scratch_types=[
            pltpu.VMEM((G,), jnp.int32),
            pltpu.VMEM((G,), jnp.int32),
            pltpu.VMEM((G,), jnp.float32),
            pltpu.VMEM((4, 128), jnp.int32),
            pltpu.VMEM((4, 128), jnp.float32),
            pltpu.VMEM((4, 128), jnp.int32),
            pltpu.VMEM((G, D), jnp.float32),
            pltpu.VMEM((ZB, D), jnp.float32),
            pltpu.VMEM((128,), jnp.int32),
            pltpu.VMEM_SHARED((CP, D), jnp.float32),
            pltpu.SemaphoreType.DMA,
            pltpu.SemaphoreType.DMA,
            pltpu.SemaphoreType.DMA,
        ],
        compiler_params=pltpu.CompilerParams(use_tc_tiling_on_sc=False,
                                             needs_layout_passes=False),
    )
    def k(col_hbm, idx_hbm, v_hbm, cnt_hbm, ego_hbm, out_hbm,
          ccol, cidx, vstg, colb, vbuf, ibuf, gbuf, zbuf, cbuf, acc,
          sem, sem_e, sem_s):
        iota16 = lax.iota(jnp.int32, 16)
        c = lax.axis_index("c")
        s = lax.axis_index("s")

        def zz(i, carry):
            for jj in range(D // 16):
                zbuf[i, pl.ds(jj * 16, 16)] = jnp.zeros((16,), jnp.float32)
            return carry
        lax.fori_loop(0, ZB, zz, 0)

        for pp in range(4):
            ch = 2 * pp + c
            for kq in range(3):
                pltpu.sync_copy(zbuf, acc.at[pl.ds(s * TS + kq * ZB, ZB)])
            pltpu.sync_copy(zbuf.at[pl.ds(0, TS - 3 * ZB)],
                            acc.at[pl.ds(s * TS + 3 * ZB, TS - 3 * ZB)])
            plsc.subcore_barrier()

            for seg in range(1):
                w = jnp.where(ch >= NCH // 2, 16, 0) + s
                pltpu.sync_copy(cnt_hbm.at[w], cbuf)
                cvec = cbuf[pl.ds(0, 16)]
                cnt = jnp.int32(0)
                for kk in range(NCH):
                    cnt = jnp.where(ch == kk, cvec[kk], cnt)
                nb = jnp.maximum((cnt + G - 1) // G, 1)

                def blk(j, carry):
                    row = (ch * 32 + w) * nbw + j
                    sds = [pltpu.async_copy(col_hbm.at[row], ccol, sem_e),
                           pltpu.async_copy(idx_hbm.at[row], cidx, sem_e),
                           pltpu.async_copy(v_hbm.at[row], vstg, sem_e)]
                    for sd in sds:
                        sd.wait()
                    base = j * G
                    for q in range(4):
                        def cmv(i, carry2):
                            sl = pl.ds(q * 128 + i * 16, 16)
                            lane = base + q * 128 + i * 16 + iota16
                            valid = lane < cnt
                            colb[q, pl.ds(i * 16, 16)] = \
                                jnp.where(valid, ccol[sl], 0)
                            return carry2
                        lax.fori_loop(0, 8, cmv, 0)
                    gds = [
                        pltpu.async_copy(ego_hbm.at[colb.at[q]],
                                         gbuf.at[pl.ds(q * 128, 128)], sem)
                        for q in range(4)
                    ]
                    for q in range(4):
                        def msk(i, carry2):
                            sl = pl.ds(q * 128 + i * 16, 16)
                            lane = base + q * 128 + i * 16 + iota16
                            valid = lane < cnt
                            so = pl.ds(i * 16, 16)
                            vbuf[q, so] = jnp.where(valid, vstg[sl], 0.0)
                            ibuf[q, so] = jnp.where(valid, cidx[sl], 0)
                            return carry2
                        lax.fori_loop(0, 8, msk, 0)
                    for gd in gds:
                        gd.wait()
                    for q in range(4):
                        def erow16(i, carry2):
                            sv = vbuf[q, pl.ds(i * 16, 16)]
                            for k16 in range(16):
                                sc = sv[k16]
                                r = q * 128 + i * 16 + k16
                                for jj in range(D // 16):
                                    sl = pl.ds(jj * 16, 16)
                                    gbuf[r, sl] = gbuf[r, sl] * sc
                            return carry2
                        lax.fori_loop(0, 8, erow16, 0)
                    scs = [
                        pltpu.async_copy(gbuf.at[pl.ds(q * 128, 128)],
                                         acc.at[ibuf.at[q]], sem_s, add=True)
                        for q in range(4)
                    ]
                    for sc2 in scs:
                        sc2.wait()
                    return carry
                lax.fori_loop(0, nb, blk, 0)

            plsc.subcore_barrier()
            base = ch * CH + s * TS

            @pl.when(s < 15)
            def _():
                pltpu.sync_copy(acc.at[pl.ds(s * TS, TS)],
                                out_hbm.at[pl.ds(base, TS)])

            @pl.when(s == 15)
            def _():
                pltpu.sync_copy(acc.at[pl.ds(s * TS, TSL)],
                                out_hbm.at[pl.ds(base, TSL)])

    return k(comp_col, comp_idx, comp_v, counts, ego)


def _dense(msg, ego, w1, b1, w2, b2, n_nodes):
    br = 1000
    grid = n_nodes // br

    def body(msg_ref, ego_ref, w1_ref, b1_ref, w2_ref, b2_ref,
             eo_ref, no_ref):
        egos = ego_ref[...]
        side = msg_ref[...] + egos
        dn = (((1,), (0,)), ((), ()))
        a = lax.dot_general(side, w1_ref[...], dn,
                            preferred_element_type=jnp.float32) + b1_ref[...]
        se = jnp.maximum(a, 0.2 * a)
        b = lax.dot_general(egos * side, w2_ref[...], dn,
                            preferred_element_type=jnp.float32) + b2_ref[...]
        be = jnp.maximum(b, 0.2 * b)
        e = se + be
        eo_ref[...] = e
        nr = jnp.sqrt(jnp.sum(e * e, axis=1, keepdims=True)) + 1e-12
        no_ref[...] = e / nr

    return pl.pallas_call(
        body,
        grid=(grid,),
        in_specs=[
            pl.BlockSpec((br, D), lambda i: (i, 0)),
            pl.BlockSpec((br, D), lambda i: (i, 0)),
            pl.BlockSpec((D, D), lambda i: (0, 0)),
            pl.BlockSpec((1, D), lambda i: (0, 0)),
            pl.BlockSpec((D, D), lambda i: (0, 0)),
            pl.BlockSpec((1, D), lambda i: (0, 0)),
        ],
        out_specs=[pl.BlockSpec((br, D), lambda i: (i, 0))] * 2,
        out_shape=(jax.ShapeDtypeStruct((n_nodes, D), jnp.float32),
                   jax.ShapeDtypeStruct((n_nodes, D), jnp.float32)),
    )(msg, ego, w1, b1.reshape(1, D), w2, b2.reshape(1, D))


def _final_gather(users, pos_items, neg_items, tables, n_user, batch):
    mesh = plsc.VectorSubcoreMesh(core_axis_name="c", subcore_axis_name="s")
    per_w = batch // 32
    width = D * len(tables)

    @functools.partial(
        pl.kernel,
        out_type=tuple(jax.ShapeDtypeStruct((batch, width), jnp.float32)
                       for _ in range(3)),
        mesh=mesh,
        scratch_types=[
            pltpu.VMEM((per_w,), jnp.int32),
            pltpu.VMEM((per_w, D), jnp.float32),
            pltpu.SemaphoreType.DMA,
        ],
        compiler_params=pltpu.CompilerParams(use_tc_tiling_on_sc=False,
                                             needs_layout_passes=False),
    )
    def k(users_h, pos_h, neg_h, t0, t1, t2, t3, o0, o1, o2,
          ibuf, gbuf, sem):
        c = lax.axis_index("c")
        s = lax.axis_index("s")
        wid = s * 2 + c
        base = wid * per_w
        for src, off, out in ((users_h, 0, o0), (pos_h, n_user, o1),
                              (neg_h, n_user, o2)):
            pltpu.sync_copy(src.at[pl.ds(base, per_w)], ibuf)
            if off:
                def addoff(i, carry):
                    sl = pl.ds(i * 16, 16)
                    ibuf[sl] = ibuf[sl] + off
                    return carry
                lax.fori_loop(0, per_w // 16, addoff, 0)
            for t, tab in enumerate((t0, t1, t2, t3)):
                pltpu.async_copy(tab.at[ibuf], gbuf, sem).wait()
                pltpu.sync_copy(gbuf, out.at[pl.ds(base, per_w),
                                             pl.ds(t * D, D)])

    return k(users, pos_items, neg_items, *tables)


def kernel(users, pos_items, neg_items, lap_rows, lap_cols, lap_vals,
           user_emb, item_emb, W1, b1, W2, b2):
    n_user = user_emb.shape[0]
    n_nodes = n_user + item_emb.shape[0]
    nnz = lap_rows.shape[0]
    half = nnz // 2
    batch = users.shape[0]
    jblk = -(-half // (16 * 4 * 128))
    jblk += jblk % 2
    ep2 = (jblk + 2) * 16 * 4 * 128
    g2 = ep2 // 128
    ew = 2 * ep2 // 32
    nbw = ew // G + 2

    def prep(a):
        a2 = a.reshape(2, half)
        return jnp.pad(a2, ((0, 0), (0, ep2 - half))).reshape(2, g2, 128)

    edata_rc = jnp.stack([prep(lap_rows), prep(lap_cols)], axis=2)
    edata_v = prep(lap_vals)

    comp_col, comp_idx, comp_v, counts = _compact(edata_rc, edata_v, g2, nbw)

    ego = jnp.concatenate([user_emb, item_emb], axis=0)
    tables = [ego]
    for l in range(len(W1)):
        msg = _spmm(comp_col, comp_idx, comp_v, counts, ego, n_nodes, nbw)
        ego, nrm = _dense(msg, ego, W1[l], b1[l], W2[l], b2[l], n_nodes)
        tables.append(nrm)
    return _final_gather(users, pos_items, neg_items, tables, n_user, batch)
